# baseline (speedup 1.0000x reference)
import functools

import jax
import jax.numpy as jnp
from jax import lax
from jax.experimental import pallas as pl
from jax.experimental.pallas import tpu as pltpu
from jax.experimental.pallas import tpu_sc as plsc

N = 10000
E = 320000
D = 128
K = 25

NC = 2
NS = 16
NW = NC * NS

C = 64
CHUNKS = 160
NBLK = 20
MB = CHUNKS // NBLK
NBUF = 4
E_PAD = NW * CHUNKS * C
NP = 10112
RPT = NP // NS

_mesh = plsc.VectorSubcoreMesh(core_axis_name="c", subcore_axis_name="s")

_f32 = jnp.float32


def _zero_fill(buf, rows, width):
    z = jnp.zeros((16,), _f32)

    def body(i, _):
        for j in range(width // 16):
            buf[i, pl.ds(j * 16, 16)] = z
        return 0

    lax.fori_loop(0, rows, body, 0)


def _zero_acc_slice(zsrc, acc, base):
    for t in range(9):
        pltpu.sync_copy(zsrc, acc.at[pl.ds(base + t * 64, 64)])
    pltpu.sync_copy(zsrc.at[pl.ds(0, 56)], acc.at[pl.ds(base + 576, 56)])


@functools.partial(
    pl.kernel,
    out_type=jax.ShapeDtypeStruct((NC, NP, D), _f32),
    mesh=_mesh,
    scratch_types=[
        pltpu.VMEM((MB, 2, C), jnp.int32),
        pltpu.VMEM((MB, 2, C), jnp.int32),
        [pltpu.VMEM((C, D), _f32) for _ in range(NBUF)],
        pltpu.VMEM_SHARED((NP, D), _f32),
        pltpu.SemaphoreType.DMA,
        pltpu.SemaphoreType.DMA,
        [pltpu.SemaphoreType.DMA for _ in range(NBUF)],
        [pltpu.SemaphoreType.DMA for _ in range(NBUF)],
    ],
)
def _spmv_kernel(y_hbm, ed, out,
                 ibuf0, ibuf1, rows, acc, is0, is1, gs, ss):
    c = lax.axis_index("c")
    s = lax.axis_index("s")
    wid = c * NS + s

    _zero_fill(rows[0], C, D)
    base = s * RPT
    _zero_acc_slice(rows[0], acc, base)

    def start_idx(b, ib, sem):
        pltpu.make_async_copy(ed.at[wid, b], ib, sem).start()

    def wait_idx(ib, sem):
        pltpu.make_async_copy(ed.at[wid, 0], ib, sem).wait()

    def start_g(ib, m, v):
        pltpu.make_async_copy(y_hbm.at[ib.at[m, 0]], rows[v], gs[v]).start()

    def wait_g(ib, m, v):
        pltpu.make_async_copy(y_hbm.at[ib.at[m, 0]], rows[v], gs[v]).wait()

    def start_s(ib, m, v):
        pltpu.async_copy(rows[v], acc.at[ib.at[m, 1]], ss[v], add=True)

    def wait_s(v):
        pltpu.make_async_copy(rows[v], acc.at[ibuf0.at[0, 1]], ss[v]).wait()

    start_idx(0, ibuf0, is0)
    plsc.subcore_barrier()

    def emit_block(b, ib, isem, ib_prev, other, osem):
        wait_idx(ib, isem)
        for m in range(MB):
            v = m % NBUF
            if m < NBUF:
                @pl.when(b > 0)
                def _():
                    wait_s(v)
            else:
                wait_s(v)
            if m == NBUF:
                @pl.when(b + 1 < NBLK)
                def _():
                    start_idx(b + 1, other, osem)
            start_g(ib, m, v)
            wv = (m - 2) % NBUF
            if m >= 2:
                wait_g(ib, m - 2, wv)
                start_s(ib, m - 2, wv)
            else:
                @pl.when(b > 0)
                def _():
                    wait_g(ib_prev, MB - 2 + m, wv)
                    start_s(ib_prev, MB - 2 + m, wv)

    def body(i, _):
        b = i * 2
        emit_block(b, ibuf0, is0, ibuf1, ibuf1, is1)
        emit_block(b + 1, ibuf1, is1, ibuf0, ibuf0, is0)
        return 0

    lax.fori_loop(0, NBLK // 2, body, 0)
    for m in (MB - 2, MB - 1):
        v = m % NBUF
        wait_g(ibuf1, m, v)
        start_s(ibuf1, m, v)
    for v in range(NBUF):
        wait_s(v)
    plsc.subcore_barrier()
    pltpu.sync_copy(acc.at[pl.ds(base, RPT)], out.at[c, pl.ds(base, RPT)])


_R = 2528
_GRID = NP // _R


def _init_body(x_ref, d_ref, w0_ref, dq_ref, y_ref, out_ref):
    deg = d_ref[0, :, 0:1] + d_ref[1, :, 0:1]
    dq = jnp.where(deg > 0, lax.rsqrt(jnp.maximum(deg, 1e-12)), 0.0)
    xv = x_ref[...]
    dq_ref[...] = dq
    y_ref[...] = dq * xv
    out_ref[...] = jnp.dot(xv, w0_ref[...], preferred_element_type=_f32)


_init_call = pl.pallas_call(
    _init_body,
    grid=(_GRID,),
    in_specs=[
        pl.BlockSpec((_R, D), lambda i: (i, 0)),
        pl.BlockSpec((NC, _R, D), lambda i: (0, i, 0)),
        pl.BlockSpec((D, D), lambda i: (0, 0)),
    ],
    out_specs=[
        pl.BlockSpec((_R, 1), lambda i: (i, 0)),
        pl.BlockSpec((_R, D), lambda i: (i, 0)),
        pl.BlockSpec((_R, D), lambda i: (i, 0)),
    ],
    out_shape=[
        jax.ShapeDtypeStruct((NP, 1), _f32),
        jax.ShapeDtypeStruct((NP, D), _f32),
        jax.ShapeDtypeStruct((NP, D), _f32),
    ],
)


def _make_combine(alpha, beta):
    def body(p_ref, txp_ref, dq_ref, w_ref, outp_ref,
             tx_ref, y_ref, out_ref):
        accv = p_ref[0] + p_ref[1]
        dq = dq_ref[...]
        tx = alpha * (dq * accv)
        if beta:
            tx = tx - txp_ref[...]
        tx_ref[...] = tx
        y_ref[...] = dq * tx
        out_ref[...] = outp_ref[...] + jnp.dot(
            tx, w_ref[...], preferred_element_type=_f32)

    return pl.pallas_call(
        body,
        grid=(_GRID,),
        in_specs=[
            pl.BlockSpec((NC, _R, D), lambda i: (0, i, 0)),
            pl.BlockSpec((_R, D), lambda i: (i, 0)),
            pl.BlockSpec((_R, 1), lambda i: (i, 0)),
            pl.BlockSpec((D, D), lambda i: (0, 0)),
            pl.BlockSpec((_R, D), lambda i: (i, 0)),
        ],
        out_specs=[
            pl.BlockSpec((_R, D), lambda i: (i, 0)),
            pl.BlockSpec((_R, D), lambda i: (i, 0)),
            pl.BlockSpec((_R, D), lambda i: (i, 0)),
        ],
        out_shape=[
            jax.ShapeDtypeStruct((NP, D), _f32),
            jax.ShapeDtypeStruct((NP, D), _f32),
            jax.ShapeDtypeStruct((NP, D), _f32),
        ],
        input_output_aliases={4: 2},
    )


_combine_first = _make_combine(-1.0, False)
_combine_rest = _make_combine(-2.0, True)


def _head_body(out_ref, bc_ref, wmu_ref, bmu_ref, wls_ref, bls_ref,
               mu_ref, ls_ref):
    h = out_ref[...] + bc_ref[...]
    h = jnp.where(h >= 0, h, 0.01 * h)
    mu_ref[...] = jnp.dot(h, wmu_ref[...], preferred_element_type=_f32) + bmu_ref[...]
    l = jnp.dot(h, wls_ref[...], preferred_element_type=_f32) + bls_ref[...]
    nrm = jnp.sqrt(jnp.sum(l * l, axis=1, keepdims=True))
    ls_ref[...] = l / jnp.maximum(nrm, 1e-12) * 1.8


_head_call = pl.pallas_call(
    _head_body,
    grid=(_GRID,),
    in_specs=[
        pl.BlockSpec((_R, D), lambda i: (i, 0)),
        pl.BlockSpec((1, D), lambda i: (0, 0)),
        pl.BlockSpec((D, D), lambda i: (0, 0)),
        pl.BlockSpec((1, D), lambda i: (0, 0)),
        pl.BlockSpec((D, D), lambda i: (0, 0)),
        pl.BlockSpec((1, D), lambda i: (0, 0)),
    ],
    out_specs=[
        pl.BlockSpec((_R, D), lambda i: (i, 0)),
        pl.BlockSpec((_R, D), lambda i: (i, 0)),
    ],
    out_shape=[
        jax.ShapeDtypeStruct((NP, D), _f32),
        jax.ShapeDtypeStruct((NP, D), _f32),
    ],
)


def kernel(x, edge_index, W_cheb, b_cheb, W_mu, b_mu, W_logstd, b_logstd):
    src = edge_index[0].astype(jnp.int32)
    dst = edge_index[1].astype(jnp.int32)
    pad = E_PAD - E
    fake = jnp.full((pad,), N, jnp.int32)
    srcp = jnp.concatenate([src, fake]).reshape(NW, CHUNKS, C)
    dstp = jnp.concatenate([dst, fake]).reshape(NW, CHUNKS, C)
    src4 = srcp.reshape(NW, NBLK, MB, C)
    dst4 = dstp.reshape(NW, NBLK, MB, C)
    ed = jnp.stack([src4, dst4], axis=3)
    ed_t = jnp.stack([dst4, src4], axis=3)
    x_pad = jnp.concatenate([x, jnp.zeros((NP - N, D), _f32)])
    ones_tab = jnp.concatenate(
        [jnp.ones((N, D), _f32), jnp.zeros((NP - N, D), _f32)])

    deg_p = _spmv_kernel(ones_tab, ed_t)
    dq, y, out = _init_call(x_pad, deg_p, W_cheb[0])

    tx_m2 = x_pad
    tx_m1 = x_pad
    for k in range(1, K):
        p = _spmv_kernel(y, ed)
        combine = _combine_first if k == 1 else _combine_rest
        tx, y, out = combine(p, tx_m2, dq, W_cheb[k], out)
        tx_m2, tx_m1 = tx_m1, tx

    mu, ls = _head_call(out, b_cheb.reshape(1, D), W_mu, b_mu.reshape(1, D),
                        W_logstd, b_logstd.reshape(1, D))
    mu = mu[:N]
    ls = ls[:N]
    return (mu, ls, mu)


# speedup vs baseline: 2.5263x; 2.5263x over previous
import functools

import jax
import jax.numpy as jnp
from jax import lax
from jax.experimental import pallas as pl
from jax.experimental.pallas import tpu as pltpu
from jax.experimental.pallas import tpu_sc as plsc

N = 10000
E = 320000
D = 128
K = 25

NC = 2
NS = 16
NW = NC * NS

C = 112
CHUNKS = 90
NBLK = 10
MB = CHUNKS // NBLK
NBUF = 3
E_PAD = NW * CHUNKS * C
NP = 10112
RPT = NP // NS

_mesh = plsc.VectorSubcoreMesh(core_axis_name="c", subcore_axis_name="s")

_f32 = jnp.float32


def _zero_fill(buf, rows, width):
    z = jnp.zeros((16,), _f32)

    def body(i, _):
        for j in range(width // 16):
            buf[i, pl.ds(j * 16, 16)] = z
        return 0

    lax.fori_loop(0, rows, body, 0)


def _zero_acc_slice(zsrc, acc, base):
    full = RPT // C
    for t in range(full):
        pltpu.sync_copy(zsrc, acc.at[pl.ds(base + t * C, C)])
    rem = RPT - full * C
    if rem:
        pltpu.sync_copy(zsrc.at[pl.ds(0, rem)],
                        acc.at[pl.ds(base + full * C, rem)])


@functools.partial(
    pl.kernel,
    out_type=jax.ShapeDtypeStruct((NC, NP, D), _f32),
    mesh=_mesh,
    scratch_types=[
        pltpu.VMEM((MB, 2, C), jnp.int32),
        pltpu.VMEM((MB, 2, C), jnp.int32),
        [pltpu.VMEM((C, D), _f32) for _ in range(NBUF)],
        pltpu.VMEM_SHARED((NP, D), _f32),
        pltpu.SemaphoreType.DMA,
        pltpu.SemaphoreType.DMA,
        [pltpu.SemaphoreType.DMA for _ in range(NBUF)],
        [pltpu.SemaphoreType.DMA for _ in range(NBUF)],
    ],
)
def _spmv_kernel(y_hbm, ed, out,
                 ibuf0, ibuf1, rows, acc, is0, is1, gs, ss):
    c = lax.axis_index("c")
    s = lax.axis_index("s")
    wid = c * NS + s

    _zero_fill(rows[0], C, D)
    base = s * RPT
    _zero_acc_slice(rows[0], acc, base)

    def start_idx(b, ib, sem):
        pltpu.make_async_copy(ed.at[wid, b], ib, sem).start()

    def wait_idx(ib, sem):
        pltpu.make_async_copy(ed.at[wid, 0], ib, sem).wait()

    def start_g(ib, m, v):
        pltpu.make_async_copy(y_hbm.at[ib.at[m, 0]], rows[v], gs[v]).start()

    def wait_g(ib, m, v):
        pltpu.make_async_copy(y_hbm.at[ib.at[m, 0]], rows[v], gs[v]).wait()

    def start_s(ib, m, v):
        pltpu.async_copy(rows[v], acc.at[ib.at[m, 1]], ss[v], add=True)

    def wait_s(v):
        pltpu.make_async_copy(rows[v], acc.at[ibuf0.at[0, 1]], ss[v]).wait()

    start_idx(0, ibuf0, is0)
    plsc.subcore_barrier()

    def emit_block(b, ib, isem, ib_prev, other, osem):
        wait_idx(ib, isem)
        for m in range(MB):
            v = m % NBUF
            if m < NBUF:
                @pl.when(b > 0)
                def _():
                    wait_s(v)
            else:
                wait_s(v)
            if m == NBUF:
                @pl.when(b + 1 < NBLK)
                def _():
                    start_idx(b + 1, other, osem)
            start_g(ib, m, v)
            wv = (m - 1) % NBUF
            if m >= 1:
                wait_g(ib, m - 1, wv)
                start_s(ib, m - 1, wv)
            else:
                @pl.when(b > 0)
                def _():
                    wait_g(ib_prev, MB - 1, wv)
                    start_s(ib_prev, MB - 1, wv)

    def body(i, _):
        b = i * 2
        emit_block(b, ibuf0, is0, ibuf1, ibuf1, is1)
        emit_block(b + 1, ibuf1, is1, ibuf0, ibuf0, is0)
        return 0

    lax.fori_loop(0, NBLK // 2, body, 0)
    m_last = MB - 1
    wait_g(ibuf1, m_last, m_last % NBUF)
    start_s(ibuf1, m_last, m_last % NBUF)
    for v in range(NBUF):
        wait_s(v)
    plsc.subcore_barrier()
    pltpu.sync_copy(acc.at[pl.ds(base, RPT)], out.at[c, pl.ds(base, RPT)])


_R = 2528
_GRID = NP // _R


def _init_body(x_ref, d_ref, w0_ref, dq_ref, y_ref, out_ref):
    deg = d_ref[0, :, 0:1] + d_ref[1, :, 0:1]
    dq = jnp.where(deg > 0, lax.rsqrt(jnp.maximum(deg, 1e-12)), 0.0)
    xv = x_ref[...]
    dq_ref[...] = dq
    y_ref[...] = dq * xv
    out_ref[...] = jnp.dot(xv, w0_ref[...], preferred_element_type=_f32)


_init_call = pl.pallas_call(
    _init_body,
    grid=(_GRID,),
    in_specs=[
        pl.BlockSpec((_R, D), lambda i: (i, 0)),
        pl.BlockSpec((NC, _R, D), lambda i: (0, i, 0)),
        pl.BlockSpec((D, D), lambda i: (0, 0)),
    ],
    out_specs=[
        pl.BlockSpec((_R, 1), lambda i: (i, 0)),
        pl.BlockSpec((_R, D), lambda i: (i, 0)),
        pl.BlockSpec((_R, D), lambda i: (i, 0)),
    ],
    out_shape=[
        jax.ShapeDtypeStruct((NP, 1), _f32),
        jax.ShapeDtypeStruct((NP, D), _f32),
        jax.ShapeDtypeStruct((NP, D), _f32),
    ],
)


def _make_combine(alpha, beta):
    def body(p_ref, txp_ref, dq_ref, w_ref, outp_ref,
             tx_ref, y_ref, out_ref):
        accv = p_ref[0] + p_ref[1]
        dq = dq_ref[...]
        tx = alpha * (dq * accv)
        if beta:
            tx = tx - txp_ref[...]
        tx_ref[...] = tx
        y_ref[...] = dq * tx
        out_ref[...] = outp_ref[...] + jnp.dot(
            tx, w_ref[...], preferred_element_type=_f32)

    return pl.pallas_call(
        body,
        grid=(_GRID,),
        in_specs=[
            pl.BlockSpec((NC, _R, D), lambda i: (0, i, 0)),
            pl.BlockSpec((_R, D), lambda i: (i, 0)),
            pl.BlockSpec((_R, 1), lambda i: (i, 0)),
            pl.BlockSpec((D, D), lambda i: (0, 0)),
            pl.BlockSpec((_R, D), lambda i: (i, 0)),
        ],
        out_specs=[
            pl.BlockSpec((_R, D), lambda i: (i, 0)),
            pl.BlockSpec((_R, D), lambda i: (i, 0)),
            pl.BlockSpec((_R, D), lambda i: (i, 0)),
        ],
        out_shape=[
            jax.ShapeDtypeStruct((NP, D), _f32),
            jax.ShapeDtypeStruct((NP, D), _f32),
            jax.ShapeDtypeStruct((NP, D), _f32),
        ],
        input_output_aliases={4: 2},
    )


_combine_first = _make_combine(-1.0, False)
_combine_rest = _make_combine(-2.0, True)


def _head_body(out_ref, bc_ref, wmu_ref, bmu_ref, wls_ref, bls_ref,
               mu_ref, ls_ref):
    h = out_ref[...] + bc_ref[...]
    h = jnp.where(h >= 0, h, 0.01 * h)
    mu_ref[...] = jnp.dot(h, wmu_ref[...], preferred_element_type=_f32) + bmu_ref[...]
    l = jnp.dot(h, wls_ref[...], preferred_element_type=_f32) + bls_ref[...]
    nrm = jnp.sqrt(jnp.sum(l * l, axis=1, keepdims=True))
    ls_ref[...] = l / jnp.maximum(nrm, 1e-12) * 1.8


_head_call = pl.pallas_call(
    _head_body,
    grid=(_GRID,),
    in_specs=[
        pl.BlockSpec((_R, D), lambda i: (i, 0)),
        pl.BlockSpec((1, D), lambda i: (0, 0)),
        pl.BlockSpec((D, D), lambda i: (0, 0)),
        pl.BlockSpec((1, D), lambda i: (0, 0)),
        pl.BlockSpec((D, D), lambda i: (0, 0)),
        pl.BlockSpec((1, D), lambda i: (0, 0)),
    ],
    out_specs=[
        pl.BlockSpec((_R, D), lambda i: (i, 0)),
        pl.BlockSpec((_R, D), lambda i: (i, 0)),
    ],
    out_shape=[
        jax.ShapeDtypeStruct((NP, D), _f32),
        jax.ShapeDtypeStruct((NP, D), _f32),
    ],
)


def kernel(x, edge_index, W_cheb, b_cheb, W_mu, b_mu, W_logstd, b_logstd):
    src = edge_index[0].astype(jnp.int32)
    dst = edge_index[1].astype(jnp.int32)
    pad = E_PAD - E
    fake = jnp.full((pad,), N, jnp.int32)
    srcp = jnp.concatenate([src, fake]).reshape(NW, CHUNKS, C)
    dstp = jnp.concatenate([dst, fake]).reshape(NW, CHUNKS, C)
    src4 = srcp.reshape(NW, NBLK, MB, C)
    dst4 = dstp.reshape(NW, NBLK, MB, C)
    ed = jnp.stack([src4, dst4], axis=3)
    ed_t = jnp.stack([dst4, src4], axis=3)
    x_pad = jnp.concatenate([x, jnp.zeros((NP - N, D), _f32)])
    ones_tab = jnp.concatenate(
        [jnp.ones((N, D), _f32), jnp.zeros((NP - N, D), _f32)])

    deg_p = _spmv_kernel(ones_tab, ed_t)
    dq, y, out = _init_call(x_pad, deg_p, W_cheb[0])

    tx_m2 = x_pad
    tx_m1 = x_pad
    for k in range(1, K):
        p = _spmv_kernel(y, ed)
        combine = _combine_first if k == 1 else _combine_rest
        tx, y, out = combine(p, tx_m2, dq, W_cheb[k], out)
        tx_m2, tx_m1 = tx_m1, tx

    mu, ls = _head_call(out, b_cheb.reshape(1, D), W_mu, b_mu.reshape(1, D),
                        W_logstd, b_logstd.reshape(1, D))
    mu = mu[:N]
    ls = ls[:N]
    return (mu, ls, mu)


# speedup vs baseline: 2.5270x; 1.0003x over previous
import functools

import jax
import jax.numpy as jnp
from jax import lax
from jax.experimental import pallas as pl
from jax.experimental.pallas import tpu as pltpu
from jax.experimental.pallas import tpu_sc as plsc

N = 10000
E = 320000
D = 128
K = 25

NC = 2
NS = 16
NW = NC * NS

C = 120
CHUNKS = 84
NBLK = 14
MB = CHUNKS // NBLK
NBUF = 3
E_PAD = NW * CHUNKS * C
NP = 10112
RPT = NP // NS

_mesh = plsc.VectorSubcoreMesh(core_axis_name="c", subcore_axis_name="s")

_f32 = jnp.float32


def _zero_fill(buf, rows, width):
    z = jnp.zeros((16,), _f32)

    def body(i, _):
        for j in range(width // 16):
            buf[i, pl.ds(j * 16, 16)] = z
        return 0

    lax.fori_loop(0, rows, body, 0)


def _zero_acc_slice(zsrc, acc, base):
    full = RPT // C
    for t in range(full):
        pltpu.sync_copy(zsrc, acc.at[pl.ds(base + t * C, C)])
    rem = RPT - full * C
    if rem:
        pltpu.sync_copy(zsrc.at[pl.ds(0, rem)],
                        acc.at[pl.ds(base + full * C, rem)])


@functools.partial(
    pl.kernel,
    out_type=jax.ShapeDtypeStruct((NC, NP, D), _f32),
    mesh=_mesh,
    scratch_types=[
        pltpu.VMEM((MB, 2, C), jnp.int32),
        pltpu.VMEM((MB, 2, C), jnp.int32),
        [pltpu.VMEM((C, D), _f32) for _ in range(NBUF)],
        pltpu.VMEM_SHARED((NP, D), _f32),
        pltpu.SemaphoreType.DMA,
        pltpu.SemaphoreType.DMA,
        [pltpu.SemaphoreType.DMA for _ in range(NBUF)],
        [pltpu.SemaphoreType.DMA for _ in range(NBUF)],
    ],
)
def _spmv_kernel(y_hbm, ed, out,
                 ibuf0, ibuf1, rows, acc, is0, is1, gs, ss):
    c = lax.axis_index("c")
    s = lax.axis_index("s")
    wid = c * NS + s

    _zero_fill(rows[0], C, D)
    base = s * RPT
    _zero_acc_slice(rows[0], acc, base)

    def start_idx(b, ib, sem):
        pltpu.make_async_copy(ed.at[wid, b], ib, sem).start()

    def wait_idx(ib, sem):
        pltpu.make_async_copy(ed.at[wid, 0], ib, sem).wait()

    def start_g(ib, m, v):
        pltpu.make_async_copy(y_hbm.at[ib.at[m, 0]], rows[v], gs[v]).start()

    def wait_g(ib, m, v):
        pltpu.make_async_copy(y_hbm.at[ib.at[m, 0]], rows[v], gs[v]).wait()

    def start_s(ib, m, v):
        pltpu.async_copy(rows[v], acc.at[ib.at[m, 1]], ss[v], add=True)

    def wait_s(v):
        pltpu.make_async_copy(rows[v], acc.at[ibuf0.at[0, 1]], ss[v]).wait()

    start_idx(0, ibuf0, is0)
    plsc.subcore_barrier()

    def emit_block(b, ib, isem, ib_prev, other, osem):
        wait_idx(ib, isem)
        for m in range(MB):
            v = m % NBUF
            if m < NBUF:
                @pl.when(b > 0)
                def _():
                    wait_s(v)
            else:
                wait_s(v)
            if m == NBUF:
                @pl.when(b + 1 < NBLK)
                def _():
                    start_idx(b + 1, other, osem)
            start_g(ib, m, v)
            wv = (m - 1) % NBUF
            if m >= 1:
                wait_g(ib, m - 1, wv)
                start_s(ib, m - 1, wv)
            else:
                @pl.when(b > 0)
                def _():
                    wait_g(ib_prev, MB - 1, wv)
                    start_s(ib_prev, MB - 1, wv)

    def body(i, _):
        b = i * 2
        emit_block(b, ibuf0, is0, ibuf1, ibuf1, is1)
        emit_block(b + 1, ibuf1, is1, ibuf0, ibuf0, is0)
        return 0

    lax.fori_loop(0, NBLK // 2, body, 0)
    m_last = MB - 1
    wait_g(ibuf1, m_last, m_last % NBUF)
    start_s(ibuf1, m_last, m_last % NBUF)
    for v in range(NBUF):
        wait_s(v)
    plsc.subcore_barrier()
    pltpu.sync_copy(acc.at[pl.ds(base, RPT)], out.at[c, pl.ds(base, RPT)])


_R = 2528
_GRID = NP // _R


def _init_body(x_ref, d_ref, w0_ref, dq_ref, y_ref, out_ref):
    deg = d_ref[0, :, 0:1] + d_ref[1, :, 0:1]
    dq = jnp.where(deg > 0, lax.rsqrt(jnp.maximum(deg, 1e-12)), 0.0)
    xv = x_ref[...]
    dq_ref[...] = dq
    y_ref[...] = dq * xv
    out_ref[...] = jnp.dot(xv, w0_ref[...], preferred_element_type=_f32)


_init_call = pl.pallas_call(
    _init_body,
    grid=(_GRID,),
    in_specs=[
        pl.BlockSpec((_R, D), lambda i: (i, 0)),
        pl.BlockSpec((NC, _R, D), lambda i: (0, i, 0)),
        pl.BlockSpec((D, D), lambda i: (0, 0)),
    ],
    out_specs=[
        pl.BlockSpec((_R, 1), lambda i: (i, 0)),
        pl.BlockSpec((_R, D), lambda i: (i, 0)),
        pl.BlockSpec((_R, D), lambda i: (i, 0)),
    ],
    out_shape=[
        jax.ShapeDtypeStruct((NP, 1), _f32),
        jax.ShapeDtypeStruct((NP, D), _f32),
        jax.ShapeDtypeStruct((NP, D), _f32),
    ],
)


def _make_combine(alpha, beta):
    def body(p_ref, txp_ref, dq_ref, w_ref, outp_ref,
             tx_ref, y_ref, out_ref):
        accv = p_ref[0] + p_ref[1]
        dq = dq_ref[...]
        tx = alpha * (dq * accv)
        if beta:
            tx = tx - txp_ref[...]
        tx_ref[...] = tx
        y_ref[...] = dq * tx
        out_ref[...] = outp_ref[...] + jnp.dot(
            tx, w_ref[...], preferred_element_type=_f32)

    return pl.pallas_call(
        body,
        grid=(_GRID,),
        in_specs=[
            pl.BlockSpec((NC, _R, D), lambda i: (0, i, 0)),
            pl.BlockSpec((_R, D), lambda i: (i, 0)),
            pl.BlockSpec((_R, 1), lambda i: (i, 0)),
            pl.BlockSpec((D, D), lambda i: (0, 0)),
            pl.BlockSpec((_R, D), lambda i: (i, 0)),
        ],
        out_specs=[
            pl.BlockSpec((_R, D), lambda i: (i, 0)),
            pl.BlockSpec((_R, D), lambda i: (i, 0)),
            pl.BlockSpec((_R, D), lambda i: (i, 0)),
        ],
        out_shape=[
            jax.ShapeDtypeStruct((NP, D), _f32),
            jax.ShapeDtypeStruct((NP, D), _f32),
            jax.ShapeDtypeStruct((NP, D), _f32),
        ],
        input_output_aliases={4: 2},
    )


_combine_first = _make_combine(-1.0, False)
_combine_rest = _make_combine(-2.0, True)


def _head_body(out_ref, bc_ref, wmu_ref, bmu_ref, wls_ref, bls_ref,
               mu_ref, ls_ref):
    h = out_ref[...] + bc_ref[...]
    h = jnp.where(h >= 0, h, 0.01 * h)
    mu_ref[...] = jnp.dot(h, wmu_ref[...], preferred_element_type=_f32) + bmu_ref[...]
    l = jnp.dot(h, wls_ref[...], preferred_element_type=_f32) + bls_ref[...]
    nrm = jnp.sqrt(jnp.sum(l * l, axis=1, keepdims=True))
    ls_ref[...] = l / jnp.maximum(nrm, 1e-12) * 1.8


_head_call = pl.pallas_call(
    _head_body,
    grid=(_GRID,),
    in_specs=[
        pl.BlockSpec((_R, D), lambda i: (i, 0)),
        pl.BlockSpec((1, D), lambda i: (0, 0)),
        pl.BlockSpec((D, D), lambda i: (0, 0)),
        pl.BlockSpec((1, D), lambda i: (0, 0)),
        pl.BlockSpec((D, D), lambda i: (0, 0)),
        pl.BlockSpec((1, D), lambda i: (0, 0)),
    ],
    out_specs=[
        pl.BlockSpec((_R, D), lambda i: (i, 0)),
        pl.BlockSpec((_R, D), lambda i: (i, 0)),
    ],
    out_shape=[
        jax.ShapeDtypeStruct((NP, D), _f32),
        jax.ShapeDtypeStruct((NP, D), _f32),
    ],
)


def kernel(x, edge_index, W_cheb, b_cheb, W_mu, b_mu, W_logstd, b_logstd):
    src = edge_index[0].astype(jnp.int32)
    dst = edge_index[1].astype(jnp.int32)
    pad = E_PAD - E
    fake = jnp.full((pad,), N, jnp.int32)
    srcp = jnp.concatenate([src, fake]).reshape(NW, CHUNKS, C)
    dstp = jnp.concatenate([dst, fake]).reshape(NW, CHUNKS, C)
    src4 = srcp.reshape(NW, NBLK, MB, C)
    dst4 = dstp.reshape(NW, NBLK, MB, C)
    ed = jnp.stack([src4, dst4], axis=3)
    ed_t = jnp.stack([dst4, src4], axis=3)
    x_pad = jnp.concatenate([x, jnp.zeros((NP - N, D), _f32)])
    ones_tab = jnp.concatenate(
        [jnp.ones((N, D), _f32), jnp.zeros((NP - N, D), _f32)])

    deg_p = _spmv_kernel(ones_tab, ed_t)
    dq, y, out = _init_call(x_pad, deg_p, W_cheb[0])

    tx_m2 = x_pad
    tx_m1 = x_pad
    for k in range(1, K):
        p = _spmv_kernel(y, ed)
        combine = _combine_first if k == 1 else _combine_rest
        tx, y, out = combine(p, tx_m2, dq, W_cheb[k], out)
        tx_m2, tx_m1 = tx_m1, tx

    mu, ls = _head_call(out, b_cheb.reshape(1, D), W_mu, b_mu.reshape(1, D),
                        W_logstd, b_logstd.reshape(1, D))
    mu = mu[:N]
    ls = ls[:N]
    return (mu, ls, mu)
